# baseline (speedup 1.0000x reference)
import functools

import jax
import jax.numpy as jnp
from jax import lax
from jax.experimental import pallas as pl
from jax.experimental.pallas import tpu as pltpu
from jax.experimental.pallas import tpu_sc as plsc

EPS = 1e-5
LANES = 128


def _sc_gather(table_c, pair_idx):
    num_rows = pair_idx.shape[0]
    info = plsc.get_sparse_core_info()
    nw = info.num_cores * info.num_subcores
    rows_per_w = num_rows // nw
    chunk = 800
    while rows_per_w % chunk:
        chunk //= 2
    n_chunks = rows_per_w // chunk

    mesh = plsc.VectorSubcoreMesh(core_axis_name="c", subcore_axis_name="s")

    @functools.partial(
        pl.kernel,
        mesh=mesh,
        out_type=jax.ShapeDtypeStruct((num_rows, LANES), jnp.float32),
        scratch_types=[
            pltpu.VMEM((chunk,), jnp.int32),
            pltpu.VMEM((chunk, LANES), jnp.float32),
            pltpu.SemaphoreType.DMA,
        ],
    )
    def gather_kernel(table_hbm, idx_hbm, out_hbm, idx_v, rows_v, sem):
        wid = lax.axis_index("s") * info.num_cores + lax.axis_index("c")
        base = wid * rows_per_w

        def body(ci, carry):
            start = pl.multiple_of(base + ci * chunk, 8)
            pltpu.sync_copy(idx_hbm.at[pl.ds(start, chunk)], idx_v)
            pltpu.async_copy(table_hbm.at[idx_v], rows_v, sem).wait()
            pltpu.sync_copy(rows_v, out_hbm.at[pl.ds(start, chunk)])
            return carry

        lax.fori_loop(0, n_chunks, body, 0)

    return gather_kernel(table_c, pair_idx)


def _tc_body(emb_ref, par_ref, pos_ref, gamma_ref, beta_ref, wp_ref, out_ref):
    e = emb_ref[...]
    p = par_ref[...] == 0
    h = jnp.where(p, e[:, :64], e[:, 64:]) + pos_ref[0]
    mu = jnp.mean(h, axis=1, keepdims=True)
    var = jnp.mean((h - mu) ** 2, axis=1, keepdims=True)
    hn = (h - mu) * lax.rsqrt(var + EPS)
    hn = hn * gamma_ref[...] + beta_ref[...]
    out_ref[...] = jnp.dot(hn, wp_ref[...], preferred_element_type=jnp.float32)


def kernel(x, table, pos_table, gamma, beta, Wp):
    b, s = x.shape
    v, d = table.shape
    m = Wp.shape[1]
    num_rows = b * s

    table_c = table.reshape(v // 2, 2 * d)

    idx_sm = jnp.swapaxes(x, 0, 1).reshape(num_rows)
    pair_idx = lax.shift_right_logical(idx_sm, 1)
    parity = (idx_sm & 1).reshape(num_rows, 1)

    emb = _sc_gather(table_c, pair_idx)

    blk = 2048
    per_s = b // blk

    out = pl.pallas_call(
        _tc_body,
        grid=(num_rows // blk,),
        in_specs=[
            pl.BlockSpec((blk, 2 * d), lambda i: (i, 0)),
            pl.BlockSpec((blk, 1), lambda i: (i, 0)),
            pl.BlockSpec((1, 1, d), lambda i, _p=per_s: (i // _p, 0, 0)),
            pl.BlockSpec((1, d), lambda i: (0, 0)),
            pl.BlockSpec((1, d), lambda i: (0, 0)),
            pl.BlockSpec((d, m), lambda i: (0, 0)),
        ],
        out_specs=pl.BlockSpec((blk, m), lambda i: (i, 0)),
        out_shape=jax.ShapeDtypeStruct((num_rows, m), jnp.float32),
    )(emb, parity, pos_table.reshape(-1, 1, d), gamma.reshape(1, d),
      beta.reshape(1, d), Wp)

    return jnp.swapaxes(out.reshape(s, b, m), 0, 1)


# speedup vs baseline: 1.1854x; 1.1854x over previous
import functools

import jax
import jax.numpy as jnp
from jax import lax
from jax.experimental import pallas as pl
from jax.experimental.pallas import tpu as pltpu
from jax.experimental.pallas import tpu_sc as plsc

EPS = 1e-5


def _sc_gather(table, idx_flat):
    num_rows = idx_flat.shape[0]
    d = table.shape[1]
    info = plsc.get_sparse_core_info()
    nw = info.num_cores * info.num_subcores
    rows_per_w = num_rows // nw
    chunk = 1280
    while rows_per_w % chunk:
        chunk //= 2
    n_chunks = rows_per_w // chunk

    mesh = plsc.VectorSubcoreMesh(core_axis_name="c", subcore_axis_name="s")

    @functools.partial(
        pl.kernel,
        mesh=mesh,
        compiler_params=pltpu.CompilerParams(use_tc_tiling_on_sc=False),
        out_type=jax.ShapeDtypeStruct((num_rows, d), jnp.float32),
        scratch_types=[
            pltpu.VMEM((chunk,), jnp.int32),
            pltpu.VMEM((chunk, d), jnp.float32),
            pltpu.SemaphoreType.DMA,
        ],
    )
    def gather_kernel(table_hbm, idx_hbm, out_hbm, idx_v, rows_v, sem):
        wid = lax.axis_index("s") * info.num_cores + lax.axis_index("c")
        base = wid * rows_per_w

        def body(ci, carry):
            start = pl.multiple_of(base + ci * chunk, 8)
            pltpu.sync_copy(idx_hbm.at[pl.ds(start, chunk)], idx_v)
            pltpu.async_copy(table_hbm.at[idx_v], rows_v, sem).wait()
            pltpu.sync_copy(rows_v, out_hbm.at[pl.ds(start, chunk)])
            return carry

        lax.fori_loop(0, n_chunks, body, 0)

    return gather_kernel(table, idx_flat)


def _tc_body(emb_ref, pos_ref, gamma_ref, beta_ref, wp_ref, out_ref):
    h = emb_ref[...] + pos_ref[0]
    mu = jnp.mean(h, axis=1, keepdims=True)
    var = jnp.mean((h - mu) ** 2, axis=1, keepdims=True)
    hn = (h - mu) * lax.rsqrt(var + EPS)
    hn = hn * gamma_ref[...] + beta_ref[...]
    out_ref[...] = jnp.dot(hn, wp_ref[...], preferred_element_type=jnp.float32)


def kernel(x, table, pos_table, gamma, beta, Wp):
    b, s = x.shape
    d = table.shape[1]
    m = Wp.shape[1]
    num_rows = b * s

    idx_sm = jnp.swapaxes(x, 0, 1).reshape(num_rows)
    emb = _sc_gather(table, idx_sm)

    blk = 2048
    per_s = b // blk

    out = pl.pallas_call(
        _tc_body,
        grid=(num_rows // blk,),
        in_specs=[
            pl.BlockSpec((blk, d), lambda i: (i, 0)),
            pl.BlockSpec((1, 1, d), lambda i, _p=per_s: (i // _p, 0, 0)),
            pl.BlockSpec((1, d), lambda i: (0, 0)),
            pl.BlockSpec((1, d), lambda i: (0, 0)),
            pl.BlockSpec((d, m), lambda i: (0, 0)),
        ],
        out_specs=pl.BlockSpec((blk, m), lambda i: (i, 0)),
        out_shape=jax.ShapeDtypeStruct((num_rows, m), jnp.float32),
    )(emb, pos_table.reshape(-1, 1, d), gamma.reshape(1, d), beta.reshape(1, d), Wp)

    return jnp.swapaxes(out.reshape(s, b, m), 0, 1)


# speedup vs baseline: 1.2113x; 1.0218x over previous
import functools

import jax
import jax.numpy as jnp
from jax import lax
from jax.experimental import pallas as pl
from jax.experimental.pallas import tpu as pltpu
from jax.experimental.pallas import tpu_sc as plsc

EPS = 1e-5


def _sc_gather(table, idx_flat):
    num_rows = idx_flat.shape[0]
    d = table.shape[1]
    info = plsc.get_sparse_core_info()
    nw = info.num_cores * info.num_subcores
    rows_per_w = num_rows // nw
    chunk = 1280
    while rows_per_w % chunk:
        chunk //= 2
    n_chunks = rows_per_w // chunk

    mesh = plsc.VectorSubcoreMesh(core_axis_name="c", subcore_axis_name="s")

    @functools.partial(
        pl.kernel,
        mesh=mesh,
        compiler_params=pltpu.CompilerParams(use_tc_tiling_on_sc=False),
        out_type=jax.ShapeDtypeStruct((num_rows, d), jnp.float32),
        scratch_types=[
            pltpu.VMEM((chunk,), jnp.int32),
            pltpu.VMEM((chunk, d), jnp.float32),
            pltpu.SemaphoreType.DMA,
        ],
    )
    def gather_kernel(table_hbm, idx_hbm, out_hbm, idx_v, rows_v, sem):
        wid = lax.axis_index("s") * info.num_cores + lax.axis_index("c")
        base = wid * rows_per_w

        def body(ci, carry):
            start = pl.multiple_of(base + ci * chunk, 8)
            pltpu.sync_copy(idx_hbm.at[pl.ds(start, chunk)], idx_v)
            pltpu.async_copy(table_hbm.at[idx_v], rows_v, sem).wait()
            pltpu.sync_copy(rows_v, out_hbm.at[pl.ds(start, chunk)])
            return carry

        lax.fori_loop(0, n_chunks, body, 0)

    return gather_kernel(table, idx_flat)


def _tc_body(emb_ref, pos_ref, gamma_ref, beta_ref, wp_ref, out_ref):
    h = emb_ref[...] + pos_ref[0]
    mu = jnp.mean(h, axis=1, keepdims=True)
    var = jnp.mean((h - mu) ** 2, axis=1, keepdims=True)
    hn = (h - mu) * lax.rsqrt(var + EPS)
    hn = hn * gamma_ref[...] + beta_ref[...]
    out_ref[...] = jnp.dot(hn, wp_ref[...], preferred_element_type=jnp.float32)


def kernel(x, table, pos_table, gamma, beta, Wp):
    b, s = x.shape
    d = table.shape[1]
    m = Wp.shape[1]
    num_rows = b * s

    idx_sm = jnp.swapaxes(x, 0, 1).reshape(num_rows)
    emb = _sc_gather(table, idx_sm)

    blk = 4096
    per_s = b // blk

    out = pl.pallas_call(
        _tc_body,
        grid=(num_rows // blk,),
        in_specs=[
            pl.BlockSpec((blk, d), lambda i: (i, 0)),
            pl.BlockSpec((1, 1, d), lambda i, _p=per_s: (i // _p, 0, 0)),
            pl.BlockSpec((1, d), lambda i: (0, 0)),
            pl.BlockSpec((1, d), lambda i: (0, 0)),
            pl.BlockSpec((d, m), lambda i: (0, 0)),
        ],
        out_specs=pl.BlockSpec((blk, m), lambda i: (i, 0)),
        out_shape=jax.ShapeDtypeStruct((num_rows, m), jnp.float32),
    )(emb, pos_table.reshape(-1, 1, d), gamma.reshape(1, d), beta.reshape(1, d), Wp)

    return jnp.swapaxes(out.reshape(s, b, m), 0, 1)


# speedup vs baseline: 1.2184x; 1.0058x over previous
import functools

import jax
import jax.numpy as jnp
from jax import lax
from jax.experimental import pallas as pl
from jax.experimental.pallas import tpu as pltpu
from jax.experimental.pallas import tpu_sc as plsc

EPS = 1e-5


def _sc_gather(table, idx_flat):
    num_rows = idx_flat.shape[0]
    d = table.shape[1]
    info = plsc.get_sparse_core_info()
    nw = info.num_cores * info.num_subcores
    rows_per_w = num_rows // nw
    chunk = 1280
    while rows_per_w % chunk:
        chunk //= 2
    n_chunks = rows_per_w // chunk

    mesh = plsc.VectorSubcoreMesh(core_axis_name="c", subcore_axis_name="s")

    @functools.partial(
        pl.kernel,
        mesh=mesh,
        compiler_params=pltpu.CompilerParams(use_tc_tiling_on_sc=False),
        out_type=jax.ShapeDtypeStruct((num_rows, d), jnp.float32),
        scratch_types=[
            pltpu.VMEM((chunk,), jnp.int32),
            pltpu.VMEM((chunk, d), jnp.float32),
            pltpu.SemaphoreType.DMA,
        ],
    )
    def gather_kernel(table_hbm, idx_hbm, out_hbm, idx_v, rows_v, sem):
        wid = lax.axis_index("s") * info.num_cores + lax.axis_index("c")
        base = wid * rows_per_w

        def body(ci, carry):
            start = pl.multiple_of(base + ci * chunk, 8)
            pltpu.sync_copy(idx_hbm.at[pl.ds(start, chunk)], idx_v)
            pltpu.async_copy(table_hbm.at[idx_v], rows_v, sem).wait()
            pltpu.sync_copy(rows_v, out_hbm.at[pl.ds(start, chunk)])
            return carry

        lax.fori_loop(0, n_chunks, body, 0)

    return gather_kernel(table, idx_flat)


def _tc_body(emb_ref, pos_ref, gamma_ref, beta_ref, wp_ref, out_ref):
    half = emb_ref.shape[0] // pos_ref.shape[0]
    for j in range(pos_ref.shape[0]):
        sl = pl.ds(j * half, half)
        h = emb_ref[sl, :] + pos_ref[j]
        mu = jnp.mean(h, axis=1, keepdims=True)
        var = jnp.mean((h - mu) ** 2, axis=1, keepdims=True)
        hn = (h - mu) * lax.rsqrt(var + EPS)
        hn = hn * gamma_ref[...] + beta_ref[...]
        out_ref[sl, :] = jnp.dot(hn, wp_ref[...], preferred_element_type=jnp.float32)


def kernel(x, table, pos_table, gamma, beta, Wp):
    b, s = x.shape
    d = table.shape[1]
    m = Wp.shape[1]
    num_rows = b * s

    idx_sm = jnp.swapaxes(x, 0, 1).reshape(num_rows)
    emb = _sc_gather(table, idx_sm)

    blk = 8192
    s_per_blk = blk // b

    out = pl.pallas_call(
        _tc_body,
        grid=(num_rows // blk,),
        in_specs=[
            pl.BlockSpec((blk, d), lambda i: (i, 0)),
            pl.BlockSpec((s_per_blk, 1, d), lambda i: (i, 0, 0)),
            pl.BlockSpec((1, d), lambda i: (0, 0)),
            pl.BlockSpec((1, d), lambda i: (0, 0)),
            pl.BlockSpec((d, m), lambda i: (0, 0)),
        ],
        out_specs=pl.BlockSpec((blk, m), lambda i: (i, 0)),
        out_shape=jax.ShapeDtypeStruct((num_rows, m), jnp.float32),
    )(emb, pos_table.reshape(-1, 1, d), gamma.reshape(1, d), beta.reshape(1, d), Wp)

    return jnp.swapaxes(out.reshape(s, b, m), 0, 1)
